# baseline (speedup 1.0000x reference)
import functools

import jax
import jax.numpy as jnp
from jax import lax
from jax.experimental import pallas as pl
from jax.experimental.pallas import tpu as pltpu
from jax.experimental.pallas import tpu_sc as plsc

_B, _N = 1024, 200
_S, _V, _D = 4, 100000, 64
_LANES = 16

_NUM_WORKERS = 32
_ITEMS = _B * _S
_ITEMS_PER_W = _ITEMS // _NUM_WORKERS
_B_PER_W = _ITEMS_PER_W // _S
_CHUNK_A, _CHUNK_B = 104, 96


def _sc_body(idx_hbm, tok_hbm, pos_hbm, out_hbm,
             pos_v, idx_a, idx_b, rows_v, gsem):
    wid = lax.axis_index("subcore") * 2 + lax.axis_index("core")

    for sv in range(_S):
        pltpu.sync_copy(pos_hbm.at[sv, pl.ds(0, _N)], pos_v.at[sv])

    def per_b(bb, carry):
        t0 = (wid * _B_PER_W + bb) * _S
        for sv in range(_S):
            t = t0 + sv
            base = pl.multiple_of(t * _N, 8)
            pltpu.sync_copy(idx_hbm.at[pl.ds(base, _CHUNK_A)], idx_a)
            pltpu.sync_copy(idx_hbm.at[pl.ds(base + _CHUNK_A, _CHUNK_B)],
                            idx_b)
            pltpu.async_copy(tok_hbm.at[idx_a],
                             rows_v.at[pl.ds(0, _CHUNK_A)], gsem).wait()
            pltpu.async_copy(tok_hbm.at[idx_b],
                             rows_v.at[pl.ds(_CHUNK_A, _CHUNK_B)], gsem).wait()

            def add_pos(n, c):
                for j in range(_D // _LANES):
                    sl = pl.ds(j * _LANES, _LANES)
                    rows_v[n, sl] = rows_v[n, sl] + pos_v[sv, n, sl]
                return c

            lax.fori_loop(0, _N, add_pos, 0)
            pltpu.sync_copy(rows_v, out_hbm.at[t])
        return carry

    lax.fori_loop(0, _B_PER_W, per_b, 0)


_sc_call = functools.partial(
    pl.kernel,
    out_type=jax.ShapeDtypeStruct((_B * _S, _N, _D), jnp.float32),
    mesh=plsc.VectorSubcoreMesh(core_axis_name="core",
                                subcore_axis_name="subcore"),
    scratch_types=[
        pltpu.VMEM((_S, _N, _D), jnp.float32),
        pltpu.VMEM((_CHUNK_A,), jnp.int32),
        pltpu.VMEM((_CHUNK_B,), jnp.int32),
        pltpu.VMEM((_N, _D), jnp.float32),
        pltpu.SemaphoreType.DMA,
    ],
    compiler_params=pltpu.CompilerParams(use_tc_tiling_on_sc=False),
)(_sc_body)


def kernel(x, token_emb, pos_emb):
    tok_flat = token_emb.reshape(_S * _V, _D)
    offs = jnp.arange(_S, dtype=jnp.int32) * _V
    idx_all = (x.astype(jnp.int32)[:, None, :] + offs[None, :, None]).reshape(-1)
    return _sc_call(idx_all, tok_flat, pos_emb)


# speedup vs baseline: 1.4086x; 1.4086x over previous
import functools

import jax
import jax.numpy as jnp
from jax import lax
from jax.experimental import pallas as pl
from jax.experimental.pallas import tpu as pltpu
from jax.experimental.pallas import tpu_sc as plsc

_B, _N = 1024, 200
_S, _V, _D = 4, 100000, 64
_LANES = 16

_NUM_WORKERS = 32
_ITEMS = _B * _S
_ITEMS_PER_W = _ITEMS // _NUM_WORKERS
_IDX_PER_W = _ITEMS_PER_W * _N
_NBUF = 4
_CHUNK_A, _CHUNK_B = 104, 96


def _sc_body(idx_hbm, tok_hbm, pos_hbm, out_hbm,
             pos_v, idx_v, rows_v, gsem, ssem):
    wid = lax.axis_index("subcore") * 2 + lax.axis_index("core")
    t0 = wid * _ITEMS_PER_W

    base0 = pl.multiple_of(t0 * _N, 8)
    pltpu.sync_copy(idx_hbm.at[pl.ds(base0, _IDX_PER_W)], idx_v)
    for sv in range(_S):
        pltpu.sync_copy(pos_hbm.at[sv, pl.ds(0, _N)], pos_v.at[sv])

    def gather_descs(i, buf):
        off = pl.multiple_of(i * _N, 8)
        ca = pltpu.make_async_copy(
            tok_hbm.at[idx_v.at[pl.ds(off, _CHUNK_A)]],
            rows_v.at[buf, pl.ds(0, _CHUNK_A)], gsem.at[buf])
        cb = pltpu.make_async_copy(
            tok_hbm.at[idx_v.at[pl.ds(off + _CHUNK_A, _CHUNK_B)]],
            rows_v.at[buf, pl.ds(_CHUNK_A, _CHUNK_B)], gsem.at[buf])
        return ca, cb

    def store_desc(t, buf):
        return pltpu.make_async_copy(rows_v.at[buf], out_hbm.at[t],
                                     ssem.at[buf])

    for i in range(2):
        ca, cb = gather_descs(i, i)
        ca.start()
        cb.start()

    def step(i2, carry):
        for par in range(_NBUF):
            i = i2 * _NBUF + par
            t = t0 + i
            ca, cb = gather_descs(i, par)
            ca.wait()
            cb.wait()
            def add_pos(n2, c):
                for h in range(2):
                    nn = n2 * 2 + h
                    for j in range(_D // _LANES):
                        sl = pl.ds(j * _LANES, _LANES)
                        rows_v[par, nn, sl] = (rows_v[par, nn, sl]
                                               + pos_v[par, nn, sl])
                return c
            lax.fori_loop(0, _N // 2, add_pos, 0)
            store_desc(t, par).start()

            nxt = i + 2
            nbuf = (par + 2) % _NBUF

            @pl.when(nxt < _ITEMS_PER_W)
            def _():
                @pl.when(i >= 2)
                def _():
                    store_desc(t, nbuf).wait()
                na, nb = gather_descs(nxt, nbuf)
                na.start()
                nb.start()
        return carry

    lax.fori_loop(0, _ITEMS_PER_W // _NBUF, step, 0)

    for par in range(_NBUF):
        store_desc(t0, par).wait()


_sc_call = functools.partial(
    pl.kernel,
    out_type=jax.ShapeDtypeStruct((_B * _S, _N, _D), jnp.float32),
    mesh=plsc.VectorSubcoreMesh(core_axis_name="core",
                                subcore_axis_name="subcore"),
    scratch_types=[
        pltpu.VMEM((_S, _N, _D), jnp.float32),
        pltpu.VMEM((_IDX_PER_W,), jnp.int32),
        pltpu.VMEM((_NBUF, _N, _D), jnp.float32),
        pltpu.SemaphoreType.DMA((_NBUF,)),
        pltpu.SemaphoreType.DMA((_NBUF,)),
    ],
    compiler_params=pltpu.CompilerParams(use_tc_tiling_on_sc=False),
)(_sc_body)


def kernel(x, token_emb, pos_emb):
    tok_flat = token_emb.reshape(_S * _V, _D)
    offs = jnp.arange(_S, dtype=jnp.int32) * _V
    idx_all = (x.astype(jnp.int32)[:, None, :] + offs[None, :, None]).reshape(-1)
    return _sc_call(idx_all, tok_flat, pos_emb)
